# baseline (speedup 1.0000x reference)
import functools

import jax
import jax.numpy as jnp
from jax import lax
from jax.experimental import pallas as pl
from jax.experimental.pallas import tpu as pltpu
from jax.experimental.pallas import tpu_sc as plsc

HIDDEN = 1024
FFN = 512
E = 64
T = 128
EPB = 2
_NEG_INF = float("-inf")

_NC = 2
_NS = 16
_LANES = 16
_NW_ACT = T // _LANES


def _logits_body(x_ref, gate_ref, out_ref):
    out_ref[...] = jax.lax.dot_general(
        gate_ref[...], x_ref[...], (((1,), (1,)), ((), ())),
        preferred_element_type=jnp.float32)


def _router_body(lg_hbm, ct_hbm, lbuf, cbuf):
    wid = lax.axis_index("s") * _NC + lax.axis_index("c")

    @pl.when(wid < _NW_ACT)
    def _():
        base = wid * _LANES
        pltpu.sync_copy(lg_hbm, lbuf)
        neg_inf = jnp.full((_LANES,), _NEG_INF, jnp.float32)
        zero_i = jnp.zeros((_LANES,), jnp.int32)
        m1, m2 = neg_inf, neg_inf
        e1, e2 = zero_i, zero_i
        for e in range(E):
            v = lbuf[pl.ds(e * T + base, _LANES)]
            ev = jnp.full((_LANES,), e, jnp.int32)
            e2 = jnp.where(v > m1, e1, jnp.where(v > m2, ev, e2))
            e1 = jnp.where(v > m1, ev, e1)
            m2 = jnp.maximum(m2, jnp.minimum(m1, v))
            m1 = jnp.maximum(m1, v)
        w1v = 1.0 / (1.0 + jnp.exp(m2 - m1))
        w2v = 1.0 - w1v
        zf = jnp.zeros((_LANES,), jnp.float32)
        for e in range(E):
            cbuf[e, :] = jnp.where(e1 == e, w1v, jnp.where(e2 == e, w2v, zf))
        pltpu.sync_copy(cbuf, ct_hbm.at[wid])


@functools.partial(
    pl.kernel,
    mesh=plsc.VectorSubcoreMesh(core_axis_name="c", subcore_axis_name="s"),
    out_type=jax.ShapeDtypeStruct((_NW_ACT, E, _LANES), jnp.float32),
    scratch_types=[
        pltpu.VMEM((E * T,), jnp.float32),
        pltpu.VMEM((E, _LANES), jnp.float32),
    ],
)
def _sc_router(lg_hbm, ct_hbm, lbuf, cbuf):
    _router_body(lg_hbm, ct_hbm, lbuf, cbuf)


def _moe_body(x_ref, c_ref, w1_ref, w2_ref, w3_ref, out_ref):
    i = pl.program_id(0)

    @pl.when(i == 0)
    def _init():
        out_ref[...] = jnp.zeros_like(out_ref)

    xb = x_ref[...]
    h1 = jax.lax.dot_general(
        xb, w1_ref[...], (((1,), (1,)), ((), ())),
        precision=jax.lax.Precision.DEFAULT,
        preferred_element_type=jnp.float32)
    h3 = jax.lax.dot_general(
        xb, w3_ref[...], (((1,), (1,)), ((), ())),
        precision=jax.lax.Precision.DEFAULT,
        preferred_element_type=jnp.float32)
    h = (h1 * jax.lax.logistic(h1)) * h3

    iota_e = jax.lax.broadcasted_iota(jnp.int32, (T, E), 1)
    c = c_ref[...]
    iota_h = jax.lax.broadcasted_iota(jnp.int32, (T, EPB * FFN), 1) // FFN
    scale = jnp.zeros((T, EPB * FFN), jnp.float32)
    for k in range(EPB):
        ck = jnp.sum(jnp.where(iota_e == i * EPB + k, c, 0.0), axis=1,
                     keepdims=True)
        scale = jnp.where(iota_h == k, ck, scale)
    out_ref[...] += jax.lax.dot_general(
        h * scale, w2_ref[...], (((1,), (0,)), ((), ())),
        precision=jax.lax.Precision.DEFAULT,
        preferred_element_type=jnp.float32)


@jax.jit
def kernel(x, gate_w, w1, w2, w3):
    lgT = pl.pallas_call(
        _logits_body,
        in_specs=[
            pl.BlockSpec((T, HIDDEN), lambda: (0, 0)),
            pl.BlockSpec((E, HIDDEN), lambda: (0, 0)),
        ],
        out_specs=pl.BlockSpec((E, T), lambda: (0, 0)),
        out_shape=jax.ShapeDtypeStruct((E, T), jnp.float32),
    )(x, gate_w)
    ct3 = _sc_router(lgT.reshape(-1))
    c = ct3.transpose(0, 2, 1).reshape(T, E)
    out = pl.pallas_call(
        _moe_body,
        grid=(E // EPB,),
        in_specs=[
            pl.BlockSpec((T, HIDDEN), lambda i: (0, 0)),
            pl.BlockSpec((T, E), lambda i: (0, 0)),
            pl.BlockSpec((EPB * FFN, HIDDEN), lambda i: (i, 0)),
            pl.BlockSpec((EPB * FFN, HIDDEN), lambda i: (i, 0)),
            pl.BlockSpec((EPB * FFN, HIDDEN), lambda i: (i, 0)),
        ],
        out_specs=pl.BlockSpec((T, HIDDEN), lambda i: (0, 0)),
        out_shape=jax.ShapeDtypeStruct((T, HIDDEN), jnp.float32),
    )(x, c, w1, w2, w3)
    return out


# speedup vs baseline: 1.0067x; 1.0067x over previous
import functools

import jax
import jax.numpy as jnp
from jax import lax
from jax.experimental import pallas as pl
from jax.experimental.pallas import tpu as pltpu
from jax.experimental.pallas import tpu_sc as plsc

HIDDEN = 1024
FFN = 512
E = 64
T = 128
EPB = 2
_NEG_INF = float("-inf")

_NC = 1
_NS = 16
_LANES = 16
_NW_ACT = T // _LANES


def _logits_body(x_ref, gate_ref, out_ref):
    out_ref[...] = jax.lax.dot_general(
        gate_ref[...], x_ref[...], (((1,), (1,)), ((), ())),
        preferred_element_type=jnp.float32)


def _router_body(lg_hbm, ct_hbm, lbuf, cbuf):
    wid = lax.axis_index("s") * _NC + lax.axis_index("c")

    @pl.when(wid < _NW_ACT)
    def _():
        base = wid * _LANES
        pltpu.sync_copy(lg_hbm, lbuf)
        neg_inf = jnp.full((_LANES,), _NEG_INF, jnp.float32)
        zero_i = jnp.zeros((_LANES,), jnp.int32)
        m1, m2 = neg_inf, neg_inf
        e1, e2 = zero_i, zero_i
        for e in range(E):
            v = lbuf[pl.ds(e * T + base, _LANES)]
            ev = jnp.full((_LANES,), e, jnp.int32)
            e2 = jnp.where(v > m1, e1, jnp.where(v > m2, ev, e2))
            e1 = jnp.where(v > m1, ev, e1)
            m2 = jnp.maximum(m2, jnp.minimum(m1, v))
            m1 = jnp.maximum(m1, v)
        w1v = 1.0 / (1.0 + jnp.exp(m2 - m1))
        w2v = 1.0 - w1v
        zf = jnp.zeros((_LANES,), jnp.float32)
        for e in range(E):
            cbuf[e, :] = jnp.where(e1 == e, w1v, jnp.where(e2 == e, w2v, zf))
        pltpu.sync_copy(cbuf, ct_hbm.at[wid])


@functools.partial(
    pl.kernel,
    mesh=plsc.VectorSubcoreMesh(core_axis_name="c", subcore_axis_name="s", num_cores=1),
    out_type=jax.ShapeDtypeStruct((_NW_ACT, E, _LANES), jnp.float32),
    scratch_types=[
        pltpu.VMEM((E * T,), jnp.float32),
        pltpu.VMEM((E, _LANES), jnp.float32),
    ],
)
def _sc_router(lg_hbm, ct_hbm, lbuf, cbuf):
    _router_body(lg_hbm, ct_hbm, lbuf, cbuf)


def _moe_body(x_ref, c_ref, w1_ref, w2_ref, w3_ref, out_ref):
    i = pl.program_id(0)

    @pl.when(i == 0)
    def _init():
        out_ref[...] = jnp.zeros_like(out_ref)

    xb = x_ref[...]
    h1 = jax.lax.dot_general(
        xb, w1_ref[...], (((1,), (1,)), ((), ())),
        precision=jax.lax.Precision.DEFAULT,
        preferred_element_type=jnp.float32)
    h3 = jax.lax.dot_general(
        xb, w3_ref[...], (((1,), (1,)), ((), ())),
        precision=jax.lax.Precision.DEFAULT,
        preferred_element_type=jnp.float32)
    h = (h1 * jax.lax.logistic(h1)) * h3

    iota_e = jax.lax.broadcasted_iota(jnp.int32, (T, E), 1)
    c = c_ref[...]
    iota_h = jax.lax.broadcasted_iota(jnp.int32, (T, EPB * FFN), 1) // FFN
    scale = jnp.zeros((T, EPB * FFN), jnp.float32)
    for k in range(EPB):
        ck = jnp.sum(jnp.where(iota_e == i * EPB + k, c, 0.0), axis=1,
                     keepdims=True)
        scale = jnp.where(iota_h == k, ck, scale)
    out_ref[...] += jax.lax.dot_general(
        h * scale, w2_ref[...], (((1,), (0,)), ((), ())),
        precision=jax.lax.Precision.DEFAULT,
        preferred_element_type=jnp.float32)


@jax.jit
def kernel(x, gate_w, w1, w2, w3):
    lgT = pl.pallas_call(
        _logits_body,
        in_specs=[
            pl.BlockSpec((T, HIDDEN), lambda: (0, 0)),
            pl.BlockSpec((E, HIDDEN), lambda: (0, 0)),
        ],
        out_specs=pl.BlockSpec((E, T), lambda: (0, 0)),
        out_shape=jax.ShapeDtypeStruct((E, T), jnp.float32),
    )(x, gate_w)
    ct3 = _sc_router(lgT.reshape(-1))
    c = ct3.transpose(0, 2, 1).reshape(T, E)
    out = pl.pallas_call(
        _moe_body,
        grid=(E // EPB,),
        in_specs=[
            pl.BlockSpec((T, HIDDEN), lambda i: (0, 0)),
            pl.BlockSpec((T, E), lambda i: (0, 0)),
            pl.BlockSpec((EPB * FFN, HIDDEN), lambda i: (i, 0)),
            pl.BlockSpec((EPB * FFN, HIDDEN), lambda i: (i, 0)),
            pl.BlockSpec((EPB * FFN, HIDDEN), lambda i: (i, 0)),
        ],
        out_specs=pl.BlockSpec((T, HIDDEN), lambda i: (0, 0)),
        out_shape=jax.ShapeDtypeStruct((T, HIDDEN), jnp.float32),
    )(x, c, w1, w2, w3)
    return out


# speedup vs baseline: 1.0105x; 1.0038x over previous
import functools

import jax
import jax.numpy as jnp
from jax import lax
from jax.experimental import pallas as pl
from jax.experimental.pallas import tpu as pltpu
from jax.experimental.pallas import tpu_sc as plsc

HIDDEN = 1024
FFN = 512
E = 64
T = 128
EPB = 2
_NEG_INF = float("-inf")

_NC = 1
_NS = 16
_LANES = 16
_NW_ACT = T // _LANES


def _logits_body(x_ref, gate_ref, out_ref):
    out_ref[...] = jax.lax.dot_general(
        gate_ref[...], x_ref[...], (((1,), (1,)), ((), ())),
        preferred_element_type=jnp.float32)


def _router_body(lg_hbm, ct_hbm, lbuf, cbuf):
    wid = lax.axis_index("s") * _NC + lax.axis_index("c")

    @pl.when(wid < _NW_ACT)
    def _():
        base = wid * _LANES
        pltpu.sync_copy(lg_hbm, lbuf)
        neg_inf = jnp.full((_LANES,), _NEG_INF, jnp.float32)
        zero_i = jnp.zeros((_LANES,), jnp.int32)
        m1, m2 = neg_inf, neg_inf
        e1, e2 = zero_i, zero_i
        for e in range(E):
            v = lbuf[pl.ds(e * T + base, _LANES)]
            ev = jnp.full((_LANES,), e, jnp.int32)
            e2 = jnp.where(v > m1, e1, jnp.where(v > m2, ev, e2))
            e1 = jnp.where(v > m1, ev, e1)
            m2 = jnp.maximum(m2, jnp.minimum(m1, v))
            m1 = jnp.maximum(m1, v)
        w1v = 1.0 / (1.0 + jnp.exp(m2 - m1))
        w2v = 1.0 - w1v
        zf = jnp.zeros((_LANES,), jnp.float32)
        for e in range(E):
            cbuf[e, :] = jnp.where(e1 == e, w1v, jnp.where(e2 == e, w2v, zf))
        pltpu.sync_copy(cbuf, ct_hbm.at[wid])


@functools.partial(
    pl.kernel,
    mesh=plsc.VectorSubcoreMesh(core_axis_name="c", subcore_axis_name="s", num_cores=1),
    out_type=jax.ShapeDtypeStruct((_NW_ACT, E, _LANES), jnp.float32),
    scratch_types=[
        pltpu.VMEM((E * T,), jnp.float32),
        pltpu.VMEM((E, _LANES), jnp.float32),
    ],
)
def _sc_router(lg_hbm, ct_hbm, lbuf, cbuf):
    _router_body(lg_hbm, ct_hbm, lbuf, cbuf)


def _moe_body(x_ref, c_ref, w1_ref, w2_ref, w3_ref, out_ref):
    i = pl.program_id(0)

    @pl.when(i == 0)
    def _init():
        out_ref[...] = jnp.zeros_like(out_ref)

    xb = x_ref[...]
    h1 = jax.lax.dot_general(
        xb, w1_ref[...], (((1,), (1,)), ((), ())),
        precision=jax.lax.Precision.DEFAULT,
        preferred_element_type=jnp.float32)
    h3 = jax.lax.dot_general(
        xb, w3_ref[...], (((1,), (1,)), ((), ())),
        precision=jax.lax.Precision.DEFAULT,
        preferred_element_type=jnp.float32)
    h = (h1 * jax.lax.logistic(h1)) * h3

    iota_e = jax.lax.broadcasted_iota(jnp.int32, (T, E), 1)
    c = c_ref[...]
    parts = []
    for k in range(EPB):
        ck = jnp.sum(jnp.where(iota_e == i * EPB + k, c, 0.0), axis=1,
                     keepdims=True)
        parts.append(h[:, k * FFN:(k + 1) * FFN] * ck)
    hs = jnp.concatenate(parts, axis=1)
    out_ref[...] += jax.lax.dot_general(
        hs, w2_ref[...], (((1,), (0,)), ((), ())),
        precision=jax.lax.Precision.DEFAULT,
        preferred_element_type=jnp.float32)


@jax.jit
def kernel(x, gate_w, w1, w2, w3):
    lgT = pl.pallas_call(
        _logits_body,
        in_specs=[
            pl.BlockSpec((T, HIDDEN), lambda: (0, 0)),
            pl.BlockSpec((E, HIDDEN), lambda: (0, 0)),
        ],
        out_specs=pl.BlockSpec((E, T), lambda: (0, 0)),
        out_shape=jax.ShapeDtypeStruct((E, T), jnp.float32),
    )(x, gate_w)
    ct3 = _sc_router(lgT.reshape(-1))
    c = ct3.transpose(0, 2, 1).reshape(T, E)
    out = pl.pallas_call(
        _moe_body,
        grid=(E // EPB,),
        in_specs=[
            pl.BlockSpec((T, HIDDEN), lambda i: (0, 0)),
            pl.BlockSpec((T, E), lambda i: (0, 0)),
            pl.BlockSpec((EPB * FFN, HIDDEN), lambda i: (i, 0)),
            pl.BlockSpec((EPB * FFN, HIDDEN), lambda i: (i, 0)),
            pl.BlockSpec((EPB * FFN, HIDDEN), lambda i: (i, 0)),
        ],
        out_specs=pl.BlockSpec((T, HIDDEN), lambda i: (0, 0)),
        out_shape=jax.ShapeDtypeStruct((T, HIDDEN), jnp.float32),
    )(x, c, w1, w2, w3)
    return out
